# baseline (speedup 1.0000x reference)
import jax
import jax.numpy as jnp
from jax.experimental import pallas as pl
from jax.experimental.pallas import tpu as pltpu

B = 8
H = 1024
S = 64
EPS = 1e-12


def _fused_kernel(x_hbm, pos_hbm, tok_hbm, gamma_ref, beta_ref, out_hbm,
                  xbuf, obuf, posbuf, tokbuf, insems, outsems, csem):
    for i in range(B):
        pltpu.make_async_copy(x_hbm.at[i], xbuf.at[i], insems.at[i]).start()
    pltpu.make_async_copy(pos_hbm, posbuf, csem).start()
    pltpu.make_async_copy(tok_hbm.at[pl.ds(1, 1)], tokbuf, csem).start()
    pltpu.make_async_copy(pos_hbm, posbuf, csem).wait()
    pltpu.make_async_copy(tok_hbm.at[pl.ds(1, 1)], tokbuf, csem).wait()

    bias = posbuf[...] + tokbuf[...]
    bias2 = jnp.concatenate([bias, bias, bias, bias], axis=0)
    gamma = gamma_ref[...]
    beta = beta_ref[...]

    for c in range(B // 4):
        for j in range(4):
            pltpu.make_async_copy(x_hbm.at[4 * c + j], xbuf.at[4 * c + j], insems.at[4 * c + j]).wait()
        e = xbuf[4 * c:4 * c + 4].reshape(4 * S, H) + bias2
        m1 = jnp.sum(e, axis=1, keepdims=True) * (1.0 / H)
        m2 = jnp.sum(e * e, axis=1, keepdims=True) * (1.0 / H)
        var = m2 - m1 * m1
        inv = jax.lax.rsqrt(var + EPS)
        obuf[4 * c:4 * c + 4] = ((e - m1) * inv * gamma + beta).reshape(4, S, H)
        for j in range(4):
            pltpu.make_async_copy(obuf.at[4 * c + j], out_hbm.at[4 * c + j], outsems.at[4 * c + j]).start()
    for i in range(B):
        pltpu.make_async_copy(obuf.at[i], out_hbm.at[i], outsems.at[i]).wait()


def kernel(input_ids, pos_table, tok_table, ln_gamma, ln_beta):
    xt = jnp.transpose(input_ids, (0, 2, 1))
    gamma2 = ln_gamma.reshape(1, H)
    beta2 = ln_beta.reshape(1, H)
    out = pl.pallas_call(
        _fused_kernel,
        in_specs=[
            pl.BlockSpec(memory_space=pl.ANY),
            pl.BlockSpec(memory_space=pl.ANY),
            pl.BlockSpec(memory_space=pl.ANY),
            pl.BlockSpec(memory_space=pltpu.MemorySpace.VMEM),
            pl.BlockSpec(memory_space=pltpu.MemorySpace.VMEM),
        ],
        out_specs=pl.BlockSpec(memory_space=pl.ANY),
        out_shape=jax.ShapeDtypeStruct((B, S, H), jnp.float32),
        scratch_shapes=[
            pltpu.VMEM((B, S, H), jnp.float32),
            pltpu.VMEM((B, S, H), jnp.float32),
            pltpu.VMEM((S, H), jnp.float32),
            pltpu.VMEM((1, H), jnp.float32),
            pltpu.SemaphoreType.DMA((B,)),
            pltpu.SemaphoreType.DMA((B,)),
            pltpu.SemaphoreType.DMA,
        ],
    )(xt, pos_table, tok_table, gamma2, beta2)
    return out
